# baseline (speedup 1.0000x reference)
import functools

import jax
import jax.numpy as jnp
from jax.experimental import pallas as pl
from jax.experimental.pallas import tpu as pltpu
from jax.experimental.pallas import tpu_sc as plsc


def _fps_body(m, pos_ref, out_ref):
    rows, cols = pos_ref.shape[2], pos_ref.shape[3]
    n = rows * cols
    px = pos_ref[0, 0]
    py = pos_ref[0, 1]
    pz = pos_ref[0, 2]
    iota = (jax.lax.broadcasted_iota(jnp.int32, (rows, cols), 0) * cols
            + jax.lax.broadcasted_iota(jnp.int32, (rows, cols), 1))
    iota_m = jax.lax.broadcasted_iota(jnp.int32, (1, m), 1)

    def pick(idx):
        msk = iota == idx
        zf = jnp.float32(0)
        lx = jnp.sum(jnp.where(msk, px, zf))
        ly = jnp.sum(jnp.where(msk, py, zf))
        lz = jnp.sum(jnp.where(msk, pz, zf))
        return lx, ly, lz

    def body(i, st):
        dd, sel, lx, ly, lz = st
        dx = px - lx
        dy = py - ly
        dz = pz - lz
        d2 = (dx * dx + dy * dy) + dz * dz
        dd = jnp.minimum(dd, d2)
        mx = jnp.max(dd)
        idx = jnp.min(jnp.where(dd == mx, iota, n))
        sel = jnp.where(iota_m == i, idx, sel)
        lx, ly, lz = pick(idx)
        return (dd, sel, lx, ly, lz)

    dd0 = jnp.full((rows, cols), 1e10, jnp.float32)
    sel0 = jnp.zeros((1, m), jnp.int32)
    lx0, ly0, lz0 = pick(0)
    _, sel, _, _, _ = jax.lax.fori_loop(1, m, body, (dd0, sel0, lx0, ly0, lz0))
    out_ref[0, :, :] = sel


def _fps_pallas(pos, m):
    B, N, _ = pos.shape
    rows = 8 if N % (8 * 128) == 0 else 1
    cols = N // rows
    pos3n = jnp.transpose(pos, (0, 2, 1)).reshape(B, 3, rows, cols)
    out = pl.pallas_call(
        functools.partial(_fps_body, m),
        grid=(B,),
        in_specs=[
            pl.BlockSpec((1, 3, rows, cols), lambda b: (b, 0, 0, 0)),
        ],
        out_specs=pl.BlockSpec((1, 1, m), lambda b: (b, 0, 0)),
        out_shape=jax.ShapeDtypeStruct((B, 1, m), jnp.int32),
        compiler_params=pltpu.CompilerParams(
            dimension_semantics=("parallel",)),
    )(pos3n)
    return out.reshape(B, m)


def _conv1d(x, p):
    y = jnp.einsum('oc,bcn->bon', p['w'], x)
    if p.get('b') is not None:
        y = y + p['b'][None, :, None]
    return y


def _conv2d(x, p):
    y = jnp.einsum('oc,bcnk->bonk', p['w'], x)
    if p.get('b') is not None:
        y = y + p['b'][None, :, None, None]
    return y


def _bn(x, p, eps=1e-5):
    axes = tuple(i for i in range(x.ndim) if i != 1)
    m = jnp.mean(x, axis=axes, keepdims=True)
    v = jnp.var(x, axis=axes, keepdims=True)
    sh = [1] * x.ndim
    sh[1] = -1
    return (x - m) / jnp.sqrt(v + eps) * p['g'].reshape(sh) + p['b'].reshape(sh)


def _topk_body(k, d_ref, o_ref):
    rows, n = d_ref.shape
    dd = d_ref[...]
    iota = jax.lax.broadcasted_iota(jnp.int32, (rows, n), 1)
    cols = []
    for _ in range(k):
        mn = jnp.min(dd, axis=1, keepdims=True)
        am = jnp.min(jnp.where(dd == mn, iota, n), axis=1, keepdims=True)
        cols.append(am)
        dd = jnp.where(iota == am, jnp.inf, dd)
    o_ref[...] = jnp.concatenate(cols, axis=1)


def _topk_smallest_idx(d2, k):
    R, N = d2.shape
    tr = 256 if R % 256 == 0 else R
    return pl.pallas_call(
        functools.partial(_topk_body, k),
        grid=(R // tr,),
        in_specs=[pl.BlockSpec((tr, N), lambda i: (i, 0))],
        out_specs=pl.BlockSpec((tr, k), lambda i: (i, 0)),
        out_shape=jax.ShapeDtypeStruct((R, k), jnp.int32),
    )(d2)


def _knn_idx(xyz, new_xyz, k):
    d = (jnp.sum(new_xyz ** 2, axis=-1)[:, :, None]
         + jnp.sum(xyz ** 2, axis=-1)[:, None, :]
         - 2.0 * jnp.einsum('bmd,bnd->bmn', new_xyz, xyz))
    B, M, N = d.shape
    return _topk_smallest_idx(d.reshape(B * M, N), k).reshape(B, M, k)


def _group(feat, idx):
    return jax.vmap(lambda f, i: f[:, i])(feat, idx)


_SC_CORES = 2
_SC_SUBCORES = 16
_SC_NW = _SC_CORES * _SC_SUBCORES


def _sc_gather_rows(table, idx):
    V, D = table.shape
    R = idx.shape[0]
    per_w = R // _SC_NW
    ch = min(128, per_w)
    n_ch = per_w // ch
    mesh = plsc.VectorSubcoreMesh(core_axis_name="c", subcore_axis_name="s")

    def body(table_hbm, idx_hbm, out_hbm, idx_v, rows_v, sem):
        wid = jax.lax.axis_index("s") * _SC_CORES + jax.lax.axis_index("c")
        base = wid * per_w

        @pl.loop(0, n_ch)
        def _(j):
            off = base + j * ch
            pltpu.sync_copy(idx_hbm.at[pl.ds(off, ch)], idx_v)
            pltpu.async_copy(table_hbm.at[idx_v], rows_v, sem).wait()
            pltpu.sync_copy(rows_v, out_hbm.at[pl.ds(off, ch)])

    return pl.kernel(
        body,
        out_type=jax.ShapeDtypeStruct((R, D), table.dtype),
        mesh=mesh,
        scratch_types=[
            pltpu.VMEM((ch,), jnp.int32),
            pltpu.VMEM((ch, D), table.dtype),
            pltpu.SemaphoreType.DMA,
        ],
    )(table, idx)


def _group_sc(feat, idx):
    B, C, N = feat.shape
    _, M, K = idx.shape
    Cp = ((C + 127) // 128) * 128
    table = jnp.transpose(feat, (0, 2, 1)).reshape(B * N, C)
    if Cp != C:
        table = jnp.pad(table, ((0, 0), (0, Cp - C)))
    flat = (idx + (jnp.arange(B, dtype=idx.dtype) * N)[:, None, None]).reshape(B * M * K)
    rows = _sc_gather_rows(table, flat)[:, :C]
    return jnp.transpose(rows.reshape(B, M, K, C), (0, 3, 1, 2))


def _query_and_group(xyz, new_xyz, feat, ns, use_xyz, idx=None):
    if idx is None:
        idx = _knn_idx(xyz, new_xyz, ns)
    gf = _group_sc(feat, idx)
    if use_xyz:
        gx = _group(jnp.transpose(xyz, (0, 2, 1)), idx) - \
            jnp.transpose(new_xyz, (0, 2, 1))[:, :, :, None]
        return jnp.concatenate([gx, gf], axis=1), idx
    return gf, idx


def _fps(xyz, m):
    def single(p):
        idxs = jnp.zeros((m,), dtype=jnp.int32)
        dists = jnp.full((p.shape[0],), 1e10, dtype=p.dtype)

        def body(i, st):
            ii, dd = st
            last = p[ii[i - 1]]
            dd = jnp.minimum(dd, jnp.sum((p - last[None, :]) ** 2, axis=-1))
            ii = ii.at[i].set(jnp.argmax(dd).astype(jnp.int32))
            return (ii, dd)

        ii, _ = jax.lax.fori_loop(1, m, body, (idxs, dists))
        return ii
    return jax.vmap(single)(xyz)


def _pt_layer(p, pos, x, ns, idx):
    C = x.shape[1]
    q = _conv1d(x, p['q'])
    k = _conv1d(x, p['k'])
    v = _conv1d(x, p['v'])
    n_kv = _group_sc(jnp.concatenate([k, v], axis=1), idx)
    gk = n_kv[:, :C]
    n_v = n_kv[:, C:]
    gx = _group(jnp.transpose(pos, (0, 2, 1)), idx) - \
        jnp.transpose(pos, (0, 2, 1))[:, :, :, None]
    r = _conv2d(gx, p['pe1'])
    r = jax.nn.relu(_bn(r, p['pe_bn']))
    n_r = _conv2d(r, p['pe2'])
    n_v = n_v + n_r
    a = q[:, :, :, None] - gk + n_r
    a = jax.nn.relu(_bn(a, p['at_bn1']))
    a = _conv2d(a, p['at1'])
    a = jax.nn.relu(_bn(a, p['at_bn2']))
    a = _conv2d(a, p['at2'])
    a = jax.nn.softmax(a, axis=-1)
    return jnp.sum(n_v * a, axis=-1)


def _pt_block(p, pos, x, ns, idx):
    y = jax.nn.relu(_bn(_conv1d(x, p['lin1']), p['bn1']))
    y = jax.nn.relu(_bn(_pt_layer(p['tr'], pos, y, ns, idx), p['bn']))
    y = _bn(_conv1d(y, p['lin2']), p['bn2'])
    return jax.nn.relu(y + x)


def _transition_down(p, pos, x, stride, ns):
    m = x.shape[-1] // stride
    idx = _fps_pallas(pos, m)
    new_pos = jax.vmap(lambda pp, ii: pp[ii])(pos, idx)
    n_x, _ = _query_and_group(pos, new_pos, x, ns, True)
    y = jax.nn.relu(_bn(_conv2d(n_x, p['c1']), p['bn1']))
    y = jax.nn.relu(_bn(_conv2d(y, p['c2']), p['bn2']))
    return new_pos, jnp.max(y, axis=-1)


def _stage(blocks, pos, x, ns):
    idx = _knn_idx(pos, pos, ns)
    for blk in blocks:
        x = _pt_block(blk, pos, x, ns, idx)
    return x


def kernel(pc, feat, params):
    pos = pc
    x = jnp.transpose(pc, (0, 2, 1))
    x = jax.nn.relu(_bn(_conv1d(x, params['in_mlp']['c1']), params['in_mlp']['bn1']))
    x = jax.nn.relu(_bn(_conv1d(x, params['in_mlp']['c2']), params['in_mlp']['bn2']))
    x = _stage(params['enc1'], pos, x, 8)
    pos, x = _transition_down(params['down1'], pos, x, 4, 16)
    x = _stage(params['enc2'], pos, x, 16)
    pos, x = _transition_down(params['down2'], pos, x, 4, 16)
    x = _stage(params['enc3'], pos, x, 16)
    pos, x = _transition_down(params['down3'], pos, x, 4, 16)
    x = _stage(params['enc4'], pos, x, 16)
    pos, x = _transition_down(params['down4'], pos, x, 4, 16)
    x = _stage(params['enc5'], pos, x, 16)
    y = jax.nn.relu(_bn(_conv1d(x, params['dec_mlp']['c1']), params['dec_mlp']['bn1']))
    y = jax.nn.relu(_bn(_conv1d(y, params['dec_mlp']['c2']), params['dec_mlp']['bn2']))
    return y


# speedup vs baseline: 1.0030x; 1.0030x over previous
import functools

import jax
import jax.numpy as jnp
from jax.experimental import pallas as pl
from jax.experimental.pallas import tpu as pltpu
from jax.experimental.pallas import tpu_sc as plsc


def _fps_body(m, pos_ref, out_ref):
    rows, cols = pos_ref.shape[2], pos_ref.shape[3]
    n = rows * cols
    px = pos_ref[0, 0]
    py = pos_ref[0, 1]
    pz = pos_ref[0, 2]
    iota = (jax.lax.broadcasted_iota(jnp.int32, (rows, cols), 0) * cols
            + jax.lax.broadcasted_iota(jnp.int32, (rows, cols), 1))
    iota_m = jax.lax.broadcasted_iota(jnp.int32, (1, m), 1)

    def pick(idx):
        msk = iota == idx
        zf = jnp.float32(0)
        lx = jnp.sum(jnp.where(msk, px, zf))
        ly = jnp.sum(jnp.where(msk, py, zf))
        lz = jnp.sum(jnp.where(msk, pz, zf))
        return lx, ly, lz

    def body(i, st):
        dd, sel, lx, ly, lz = st
        dx = px - lx
        dy = py - ly
        dz = pz - lz
        d2 = (dx * dx + dy * dy) + dz * dz
        dd = jnp.minimum(dd, d2)
        mx = jnp.max(dd)
        idx = jnp.min(jnp.where(dd == mx, iota, n))
        sel = jnp.where(iota_m == i, idx, sel)
        lx, ly, lz = pick(idx)
        return (dd, sel, lx, ly, lz)

    dd0 = jnp.full((rows, cols), 1e10, jnp.float32)
    sel0 = jnp.zeros((1, m), jnp.int32)
    lx0, ly0, lz0 = pick(0)
    _, sel, _, _, _ = jax.lax.fori_loop(1, m, body, (dd0, sel0, lx0, ly0, lz0),
                                        unroll=4)
    out_ref[0, :, :] = sel


def _fps_pallas(pos, m):
    B, N, _ = pos.shape
    rows = 8 if N % (8 * 128) == 0 else 1
    cols = N // rows
    pos3n = jnp.transpose(pos, (0, 2, 1)).reshape(B, 3, rows, cols)
    out = pl.pallas_call(
        functools.partial(_fps_body, m),
        grid=(B,),
        in_specs=[
            pl.BlockSpec((1, 3, rows, cols), lambda b: (b, 0, 0, 0)),
        ],
        out_specs=pl.BlockSpec((1, 1, m), lambda b: (b, 0, 0)),
        out_shape=jax.ShapeDtypeStruct((B, 1, m), jnp.int32),
        compiler_params=pltpu.CompilerParams(
            dimension_semantics=("parallel",)),
    )(pos3n)
    return out.reshape(B, m)


def _group(feat, idx):
    return jax.vmap(lambda f, i: f[:, i])(feat, idx)


def _conv1d(x, p):
    y = jnp.einsum('oc,bcn->bon', p['w'], x)
    if p.get('b') is not None:
        y = y + p['b'][None, :, None]
    return y


def _conv2d(x, p):
    y = jnp.einsum('oc,bcnk->bonk', p['w'], x)
    if p.get('b') is not None:
        y = y + p['b'][None, :, None, None]
    return y


def _bn(x, p, eps=1e-5):
    axes = tuple(i for i in range(x.ndim) if i != 1)
    m = jnp.mean(x, axis=axes, keepdims=True)
    v = jnp.var(x, axis=axes, keepdims=True)
    sh = [1] * x.ndim
    sh[1] = -1
    return (x - m) / jnp.sqrt(v + eps) * p['g'].reshape(sh) + p['b'].reshape(sh)


def _topk_body(k, d_ref, o_ref):
    rows, n = d_ref.shape
    dd = d_ref[...]
    iota = jax.lax.broadcasted_iota(jnp.int32, (rows, n), 1)
    cols = []
    for _ in range(k):
        mn = jnp.min(dd, axis=1, keepdims=True)
        am = jnp.min(jnp.where(dd == mn, iota, n), axis=1, keepdims=True)
        cols.append(am)
        dd = jnp.where(iota == am, jnp.inf, dd)
    o_ref[...] = jnp.concatenate(cols, axis=1)


def _topk_smallest_idx(d2, k):
    R, N = d2.shape
    tr = 256 if R % 256 == 0 else R
    return pl.pallas_call(
        functools.partial(_topk_body, k),
        grid=(R // tr,),
        in_specs=[pl.BlockSpec((tr, N), lambda i: (i, 0))],
        out_specs=pl.BlockSpec((tr, k), lambda i: (i, 0)),
        out_shape=jax.ShapeDtypeStruct((R, k), jnp.int32),
    )(d2)


def _knn_idx(xyz, new_xyz, k):
    d = (jnp.sum(new_xyz ** 2, axis=-1)[:, :, None]
         + jnp.sum(xyz ** 2, axis=-1)[:, None, :]
         - 2.0 * jnp.einsum('bmd,bnd->bmn', new_xyz, xyz))
    B, M, N = d.shape
    return _topk_smallest_idx(d.reshape(B * M, N), k).reshape(B, M, k)


_SC_CORES = 2
_SC_SUBCORES = 16
_SC_NW = _SC_CORES * _SC_SUBCORES


def _sc_gather_rows(table, idx):
    V, D = table.shape
    R = idx.shape[0]
    per_w = R // _SC_NW
    ch = min(128, per_w)
    n_ch = per_w // ch
    mesh = plsc.VectorSubcoreMesh(core_axis_name="c", subcore_axis_name="s")

    def body(table_hbm, idx_hbm, out_hbm, idx_v, rows_v, sem):
        wid = jax.lax.axis_index("s") * _SC_CORES + jax.lax.axis_index("c")
        base = wid * per_w

        @pl.loop(0, n_ch)
        def _(j):
            off = base + j * ch
            pltpu.sync_copy(idx_hbm.at[pl.ds(off, ch)], idx_v)
            pltpu.async_copy(table_hbm.at[idx_v], rows_v, sem).wait()
            pltpu.sync_copy(rows_v, out_hbm.at[pl.ds(off, ch)])

    return pl.kernel(
        body,
        out_type=jax.ShapeDtypeStruct((R, D), table.dtype),
        mesh=mesh,
        scratch_types=[
            pltpu.VMEM((ch,), jnp.int32),
            pltpu.VMEM((ch, D), table.dtype),
            pltpu.SemaphoreType.DMA,
        ],
    )(table, idx)


def _group_sc(feat, idx):
    B, C, N = feat.shape
    _, M, K = idx.shape
    Cp = ((C + 127) // 128) * 128
    table = jnp.transpose(feat, (0, 2, 1)).reshape(B * N, C)
    if Cp != C:
        table = jnp.pad(table, ((0, 0), (0, Cp - C)))
    flat = (idx + (jnp.arange(B, dtype=idx.dtype) * N)[:, None, None]).reshape(B * M * K)
    rows = _sc_gather_rows(table, flat)[:, :C]
    return jnp.transpose(rows.reshape(B, M, K, C), (0, 3, 1, 2))


def _pt_layer(p, pos, x, ns, idx):
    C = x.shape[1]
    q = _conv1d(x, p['q'])
    k = _conv1d(x, p['k'])
    v = _conv1d(x, p['v'])
    n_kv = _group_sc(jnp.concatenate([k, v], axis=1), idx)
    gk = n_kv[:, :C]
    n_v = n_kv[:, C:]
    gx = _group(jnp.transpose(pos, (0, 2, 1)), idx) - \
        jnp.transpose(pos, (0, 2, 1))[:, :, :, None]
    r = _conv2d(gx, p['pe1'])
    r = jax.nn.relu(_bn(r, p['pe_bn']))
    n_r = _conv2d(r, p['pe2'])
    n_v = n_v + n_r
    a = q[:, :, :, None] - gk + n_r
    a = jax.nn.relu(_bn(a, p['at_bn1']))
    a = _conv2d(a, p['at1'])
    a = jax.nn.relu(_bn(a, p['at_bn2']))
    a = _conv2d(a, p['at2'])
    a = jax.nn.softmax(a, axis=-1)
    return jnp.sum(n_v * a, axis=-1)


def _pt_block(p, pos, x, ns, idx):
    y = jax.nn.relu(_bn(_conv1d(x, p['lin1']), p['bn1']))
    y = jax.nn.relu(_bn(_pt_layer(p['tr'], pos, y, ns, idx), p['bn']))
    y = _bn(_conv1d(y, p['lin2']), p['bn2'])
    return jax.nn.relu(y + x)


def _transition_down(p, pos, x, stride, ns):
    m = x.shape[-1] // stride
    idx = _fps_pallas(pos, m)
    new_pos = jax.vmap(lambda pp, ii: pp[ii])(pos, idx)
    nn = _knn_idx(pos, new_pos, ns)
    gf = _group_sc(x, nn)
    gx = _group(jnp.transpose(pos, (0, 2, 1)), nn) - \
        jnp.transpose(new_pos, (0, 2, 1))[:, :, :, None]
    n_x = jnp.concatenate([gx, gf], axis=1)
    y = jax.nn.relu(_bn(_conv2d(n_x, p['c1']), p['bn1']))
    y = jax.nn.relu(_bn(_conv2d(y, p['c2']), p['bn2']))
    return new_pos, jnp.max(y, axis=-1)


def _stage(blocks, pos, x, ns):
    idx = _knn_idx(pos, pos, ns)
    for blk in blocks:
        x = _pt_block(blk, pos, x, ns, idx)
    return x


def kernel(pc, feat, params):
    pos = pc
    x = jnp.transpose(pc, (0, 2, 1))
    x = jax.nn.relu(_bn(_conv1d(x, params['in_mlp']['c1']), params['in_mlp']['bn1']))
    x = jax.nn.relu(_bn(_conv1d(x, params['in_mlp']['c2']), params['in_mlp']['bn2']))
    x = _stage(params['enc1'], pos, x, 8)
    pos, x = _transition_down(params['down1'], pos, x, 4, 16)
    x = _stage(params['enc2'], pos, x, 16)
    pos, x = _transition_down(params['down2'], pos, x, 4, 16)
    x = _stage(params['enc3'], pos, x, 16)
    pos, x = _transition_down(params['down3'], pos, x, 4, 16)
    x = _stage(params['enc4'], pos, x, 16)
    pos, x = _transition_down(params['down4'], pos, x, 4, 16)
    x = _stage(params['enc5'], pos, x, 16)
    y = jax.nn.relu(_bn(_conv1d(x, params['dec_mlp']['c1']), params['dec_mlp']['bn1']))
    y = jax.nn.relu(_bn(_conv1d(y, params['dec_mlp']['c2']), params['dec_mlp']['bn2']))
    return y


# speedup vs baseline: 1.0674x; 1.0641x over previous
import functools

import jax
import jax.numpy as jnp
from jax.experimental import pallas as pl
from jax.experimental.pallas import tpu as pltpu
from jax.experimental.pallas import tpu_sc as plsc


def _fps_body(m, pos_ref, sm_ref, out_ref):
    rows, cols = pos_ref.shape[2], pos_ref.shape[3]
    n = rows * cols
    px = pos_ref[0, 0]
    py = pos_ref[0, 1]
    pz = pos_ref[0, 2]
    iota = (jax.lax.broadcasted_iota(jnp.int32, (rows, cols), 0) * cols
            + jax.lax.broadcasted_iota(jnp.int32, (rows, cols), 1))
    out_ref[0, 0, 0] = jnp.int32(0)

    def body(i, st):
        dd, lx, ly, lz = st
        dx = px - lx
        dy = py - ly
        dz = pz - lz
        d2 = (dx * dx + dy * dy) + dz * dz
        dd = jnp.minimum(dd, d2)
        mx = jnp.max(dd)
        idx = jnp.min(jnp.where(dd == mx, iota, n))
        out_ref[0, 0, i] = idx
        return (dd, sm_ref[0, 0, idx], sm_ref[0, 1, idx], sm_ref[0, 2, idx])

    dd0 = jnp.full((rows, cols), 1e10, jnp.float32)
    jax.lax.fori_loop(1, m, body,
                      (dd0, sm_ref[0, 0, 0], sm_ref[0, 1, 0], sm_ref[0, 2, 0]),
                      unroll=4)


def _fps_pallas(pos, m):
    B, N, _ = pos.shape
    rows = 8 if N % (8 * 128) == 0 else 1
    cols = N // rows
    pos3n = jnp.transpose(pos, (0, 2, 1))
    out = pl.pallas_call(
        functools.partial(_fps_body, m),
        grid=(B,),
        in_specs=[
            pl.BlockSpec((1, 3, rows, cols), lambda b: (b, 0, 0, 0)),
            pl.BlockSpec((1, 3, N), lambda b: (b, 0, 0),
                         memory_space=pltpu.SMEM),
        ],
        out_specs=pl.BlockSpec((1, 1, m), lambda b: (b, 0, 0),
                               memory_space=pltpu.SMEM),
        out_shape=jax.ShapeDtypeStruct((B, 1, m), jnp.int32),
        compiler_params=pltpu.CompilerParams(
            dimension_semantics=("parallel",)),
    )(pos3n.reshape(B, 3, rows, cols), pos3n)
    return out.reshape(B, m)


def _group(feat, idx):
    return jax.vmap(lambda f, i: f[:, i])(feat, idx)


def _conv1d(x, p):
    y = jnp.einsum('oc,bcn->bon', p['w'], x)
    if p.get('b') is not None:
        y = y + p['b'][None, :, None]
    return y


def _conv2d(x, p):
    y = jnp.einsum('oc,bcnk->bonk', p['w'], x)
    if p.get('b') is not None:
        y = y + p['b'][None, :, None, None]
    return y


def _bn(x, p, eps=1e-5):
    axes = tuple(i for i in range(x.ndim) if i != 1)
    m = jnp.mean(x, axis=axes, keepdims=True)
    v = jnp.var(x, axis=axes, keepdims=True)
    sh = [1] * x.ndim
    sh[1] = -1
    return (x - m) / jnp.sqrt(v + eps) * p['g'].reshape(sh) + p['b'].reshape(sh)


def _topk_body(k, d_ref, o_ref):
    rows, n = d_ref.shape
    dd = d_ref[...]
    iota = jax.lax.broadcasted_iota(jnp.int32, (rows, n), 1)
    cols = []
    for _ in range(k):
        mn = jnp.min(dd, axis=1, keepdims=True)
        am = jnp.min(jnp.where(dd == mn, iota, n), axis=1, keepdims=True)
        cols.append(am)
        dd = jnp.where(iota == am, jnp.inf, dd)
    o_ref[...] = jnp.concatenate(cols, axis=1)


def _topk_smallest_idx(d2, k):
    R, N = d2.shape
    tr = 256 if R % 256 == 0 else R
    return pl.pallas_call(
        functools.partial(_topk_body, k),
        grid=(R // tr,),
        in_specs=[pl.BlockSpec((tr, N), lambda i: (i, 0))],
        out_specs=pl.BlockSpec((tr, k), lambda i: (i, 0)),
        out_shape=jax.ShapeDtypeStruct((R, k), jnp.int32),
    )(d2)


def _knn_idx(xyz, new_xyz, k):
    d = (jnp.sum(new_xyz ** 2, axis=-1)[:, :, None]
         + jnp.sum(xyz ** 2, axis=-1)[:, None, :]
         - 2.0 * jnp.einsum('bmd,bnd->bmn', new_xyz, xyz))
    B, M, N = d.shape
    return _topk_smallest_idx(d.reshape(B * M, N), k).reshape(B, M, k)


_SC_CORES = 2
_SC_SUBCORES = 16
_SC_NW = _SC_CORES * _SC_SUBCORES


def _sc_gather_rows(table, idx):
    V, D = table.shape
    R = idx.shape[0]
    per_w = R // _SC_NW
    ch = min(128, per_w)
    n_ch = per_w // ch
    mesh = plsc.VectorSubcoreMesh(core_axis_name="c", subcore_axis_name="s")

    def body(table_hbm, idx_hbm, out_hbm, idx_v, rows_v, sem):
        wid = jax.lax.axis_index("s") * _SC_CORES + jax.lax.axis_index("c")
        base = wid * per_w

        @pl.loop(0, n_ch)
        def _(j):
            off = base + j * ch
            pltpu.sync_copy(idx_hbm.at[pl.ds(off, ch)], idx_v)
            pltpu.async_copy(table_hbm.at[idx_v], rows_v, sem).wait()
            pltpu.sync_copy(rows_v, out_hbm.at[pl.ds(off, ch)])

    return pl.kernel(
        body,
        out_type=jax.ShapeDtypeStruct((R, D), table.dtype),
        mesh=mesh,
        scratch_types=[
            pltpu.VMEM((ch,), jnp.int32),
            pltpu.VMEM((ch, D), table.dtype),
            pltpu.SemaphoreType.DMA,
        ],
    )(table, idx)


def _group_sc(feat, idx):
    B, C, N = feat.shape
    _, M, K = idx.shape
    Cp = ((C + 127) // 128) * 128
    table = jnp.transpose(feat, (0, 2, 1)).reshape(B * N, C)
    if Cp != C:
        table = jnp.pad(table, ((0, 0), (0, Cp - C)))
    flat = (idx + (jnp.arange(B, dtype=idx.dtype) * N)[:, None, None]).reshape(B * M * K)
    rows = _sc_gather_rows(table, flat)[:, :C]
    return jnp.transpose(rows.reshape(B, M, K, C), (0, 3, 1, 2))


def _pt_layer(p, pos, x, ns, idx):
    C = x.shape[1]
    q = _conv1d(x, p['q'])
    k = _conv1d(x, p['k'])
    v = _conv1d(x, p['v'])
    n_kv = _group_sc(jnp.concatenate([k, v], axis=1), idx)
    gk = n_kv[:, :C]
    n_v = n_kv[:, C:]
    gx = _group(jnp.transpose(pos, (0, 2, 1)), idx) - \
        jnp.transpose(pos, (0, 2, 1))[:, :, :, None]
    r = _conv2d(gx, p['pe1'])
    r = jax.nn.relu(_bn(r, p['pe_bn']))
    n_r = _conv2d(r, p['pe2'])
    n_v = n_v + n_r
    a = q[:, :, :, None] - gk + n_r
    a = jax.nn.relu(_bn(a, p['at_bn1']))
    a = _conv2d(a, p['at1'])
    a = jax.nn.relu(_bn(a, p['at_bn2']))
    a = _conv2d(a, p['at2'])
    a = jax.nn.softmax(a, axis=-1)
    return jnp.sum(n_v * a, axis=-1)


def _pt_block(p, pos, x, ns, idx):
    y = jax.nn.relu(_bn(_conv1d(x, p['lin1']), p['bn1']))
    y = jax.nn.relu(_bn(_pt_layer(p['tr'], pos, y, ns, idx), p['bn']))
    y = _bn(_conv1d(y, p['lin2']), p['bn2'])
    return jax.nn.relu(y + x)


def _transition_down(p, pos, x, stride, ns):
    m = x.shape[-1] // stride
    idx = _fps_pallas(pos, m)
    new_pos = jax.vmap(lambda pp, ii: pp[ii])(pos, idx)
    nn = _knn_idx(pos, new_pos, ns)
    gf = _group_sc(x, nn)
    gx = _group(jnp.transpose(pos, (0, 2, 1)), nn) - \
        jnp.transpose(new_pos, (0, 2, 1))[:, :, :, None]
    n_x = jnp.concatenate([gx, gf], axis=1)
    y = jax.nn.relu(_bn(_conv2d(n_x, p['c1']), p['bn1']))
    y = jax.nn.relu(_bn(_conv2d(y, p['c2']), p['bn2']))
    return new_pos, jnp.max(y, axis=-1)


def _stage(blocks, pos, x, ns):
    idx = _knn_idx(pos, pos, ns)
    for blk in blocks:
        x = _pt_block(blk, pos, x, ns, idx)
    return x


def kernel(pc, feat, params):
    pos = pc
    x = jnp.transpose(pc, (0, 2, 1))
    x = jax.nn.relu(_bn(_conv1d(x, params['in_mlp']['c1']), params['in_mlp']['bn1']))
    x = jax.nn.relu(_bn(_conv1d(x, params['in_mlp']['c2']), params['in_mlp']['bn2']))
    x = _stage(params['enc1'], pos, x, 8)
    pos, x = _transition_down(params['down1'], pos, x, 4, 16)
    x = _stage(params['enc2'], pos, x, 16)
    pos, x = _transition_down(params['down2'], pos, x, 4, 16)
    x = _stage(params['enc3'], pos, x, 16)
    pos, x = _transition_down(params['down3'], pos, x, 4, 16)
    x = _stage(params['enc4'], pos, x, 16)
    pos, x = _transition_down(params['down4'], pos, x, 4, 16)
    x = _stage(params['enc5'], pos, x, 16)
    y = jax.nn.relu(_bn(_conv1d(x, params['dec_mlp']['c1']), params['dec_mlp']['bn1']))
    y = jax.nn.relu(_bn(_conv1d(y, params['dec_mlp']['c2']), params['dec_mlp']['bn2']))
    return y
